# Initial kernel scaffold; baseline (speedup 1.0000x reference)
import jax
import jax.numpy as jnp
from jax.experimental import pallas as pl


def kernel(radar_voxel_feat, lidar_voxel_coords, radar_voxel_coords):
    raise NotImplementedError("write your pallas kernel here")


# speedup vs baseline: 1.5696x; 1.5696x over previous
import jax
import jax.numpy as jnp
from jax.experimental import pallas as pl

V = 16384
N = 4096
VB = 1024
GRID = V // VB


def _nn_body(af_ref, bf_ref, ft_ref, o_ref):
    ab = jnp.dot(af_ref[...], bf_ref[...], preferred_element_type=jnp.float32)

    af = af_ref[...]
    a2 = jnp.sum(af * af, axis=1, keepdims=True)
    bf = bf_ref[...]
    b2 = jnp.sum(bf * bf, axis=0, keepdims=True)

    d2 = jnp.maximum(a2 + b2 - 2.0 * ab, 0.0)
    v = jnp.sqrt(d2)
    minval = jnp.min(v, axis=1, keepdims=True)
    iota = jax.lax.broadcasted_iota(jnp.int32, v.shape, 1)
    idx = jnp.min(jnp.where(v == minval, iota, jnp.int32(N)),
                  axis=1, keepdims=True)

    sal = 0.6 * jnp.abs(ft_ref[0:1]) + 0.4 * ft_ref[1:2]
    w = jnp.min(jnp.where(idx == iota,
                          jnp.broadcast_to(sal, v.shape),
                          jnp.float32(jnp.inf)), axis=1)
    o_ref[...] = jax.nn.sigmoid(w).reshape(1, 1, VB)


def kernel(radar_voxel_feat, lidar_voxel_coords, radar_voxel_coords):
    lf = lidar_voxel_coords.astype(jnp.float32)
    rf = radar_voxel_coords.astype(jnp.float32)
    af = lf
    bf = rf.T
    ft = radar_voxel_feat[:, :2].T

    out = pl.pallas_call(
        _nn_body,
        grid=(GRID,),
        in_specs=[
            pl.BlockSpec((VB, 3), lambda i: (i, 0)),
            pl.BlockSpec((3, N), lambda i: (0, 0)),
            pl.BlockSpec((2, N), lambda i: (0, 0)),
        ],
        out_specs=pl.BlockSpec((1, 1, VB), lambda i: (i, 0, 0)),
        out_shape=jax.ShapeDtypeStruct((GRID, 1, VB), jnp.float32),
    )(af, bf, ft)
    return out.reshape(V)
